# baseline (speedup 1.0000x reference)
import functools

import jax
import jax.numpy as jnp
from jax import lax
from jax.experimental import pallas as pl
from jax.experimental.pallas import tpu as pltpu
from jax.experimental.pallas import tpu_sc as plsc

_NC = 2
_NS = 16
_NW = _NC * _NS

_CHUNK_ROWS = 32


def _sc_copy(rows, d):
    rows_per_w = rows // _NW
    n_chunks = rows_per_w // _CHUNK_ROWS
    mesh = plsc.VectorSubcoreMesh(core_axis_name="c", subcore_axis_name="s")

    @functools.partial(
        pl.kernel,
        out_type=jax.ShapeDtypeStruct((rows, d), jnp.float32),
        mesh=mesh,
        scratch_types=[
            pltpu.VMEM((_CHUNK_ROWS, d), jnp.float32),
            pltpu.SemaphoreType.DMA,
        ],
    )
    def k(x_hbm, o_hbm, buf, sem):
        wid = lax.axis_index("s") * _NC + lax.axis_index("c")
        base = wid * rows_per_w

        def step(i, carry):
            off = base + i * _CHUNK_ROWS
            pltpu.async_copy(x_hbm.at[pl.ds(off, _CHUNK_ROWS)], buf, sem).wait()
            pltpu.async_copy(buf, o_hbm.at[pl.ds(off, _CHUNK_ROWS)], sem).wait()
            return carry

        lax.fori_loop(0, n_chunks, step, 0)

    return k


def kernel(x, gate_w, gate_b, w1, b1, w2, b2):
    b, n, d = x.shape
    x2 = x.reshape(b * n, d)
    out = _sc_copy(b * n, d)(x2)
    return out.reshape(b, n, d)
